# Initial kernel scaffold; baseline (speedup 1.0000x reference)
import jax
import jax.numpy as jnp
from jax.experimental import pallas as pl


def kernel(seq, token_table, pos_table):
    raise NotImplementedError("write your pallas kernel here")


# speedup vs baseline: 3.6882x; 3.6882x over previous
import functools

import jax
import jax.numpy as jnp
from jax import lax
from jax.experimental import pallas as pl
from jax.experimental.pallas import tpu as pltpu
from jax.experimental.pallas import tpu_sc as plsc

VOCAB = 100000
MAX_LEN = 200
DEPTH = 64
BATCH = 4096
SEQ_LEN = 200

NC, NS, L = 2, 16, 16
NW = NC * NS
TOTAL = BATCH * SEQ_LEN
GU = 100
GPC = 8
CHUNK = GU * GPC
ROWS_PER_CHUNK = CHUNK // SEQ_LEN
N_CHUNKS_PER_W = TOTAL // (NW * CHUNK)
GU_PER_W = N_CHUNKS_PER_W * GPC
DL = DEPTH // L


@functools.partial(
    pl.kernel,
    mesh=plsc.VectorSubcoreMesh(core_axis_name="c", subcore_axis_name="s"),
    out_type=jax.ShapeDtypeStruct((TOTAL, DEPTH), jnp.float32),
    scratch_types=[
        pltpu.VMEM((SEQ_LEN, DEPTH), jnp.float32),
        pltpu.VMEM((GPC, GU), jnp.int32),
        pltpu.VMEM((CHUNK, DEPTH), jnp.float32),
        pltpu.SemaphoreType.DMA,
    ],
    compiler_params=pltpu.CompilerParams(use_tc_tiling_on_sc=False),
)
def _emb(tok_hbm, seq_hbm, pos_hbm, out_hbm, pos_v, idx_v, rows_v, sem):
    wid = lax.axis_index("s") * NC + lax.axis_index("c")
    pltpu.sync_copy(pos_hbm, pos_v)

    def chunk_body(ci, carry):
        g0 = wid * GU_PER_W + ci * GPC
        pltpu.sync_copy(seq_hbm.at[pl.ds(g0, GPC)], idx_v)
        copies = [
            pltpu.async_copy(
                tok_hbm.at[idx_v.at[j]],
                rows_v.at[pl.ds(j * GU, GU)],
                sem,
            )
            for j in range(GPC)
        ]
        for c in copies:
            c.wait()

        def add_body(p, c2):
            pv = [pos_v[p, pl.ds(j * L, L)] for j in range(DL)]
            for r in range(ROWS_PER_CHUNK):
                row = r * SEQ_LEN + p
                for j in range(DL):
                    rows_v[row, pl.ds(j * L, L)] = (
                        rows_v[row, pl.ds(j * L, L)] + pv[j]
                    )
            return c2

        lax.fori_loop(0, SEQ_LEN, add_body, 0)
        flat0 = (wid * N_CHUNKS_PER_W + ci) * CHUNK
        pltpu.sync_copy(rows_v, out_hbm.at[pl.ds(flat0, CHUNK)])
        return carry

    lax.fori_loop(0, N_CHUNKS_PER_W, chunk_body, 0)


def kernel(seq, token_table, pos_table):
    seq_flat = seq.reshape(-1).astype(jnp.int32).reshape(TOTAL // GU, GU)
    pos = pos_table[:SEQ_LEN].astype(jnp.float32)
    out = _emb(token_table.astype(jnp.float32), seq_flat, pos)
    return out.reshape(BATCH, SEQ_LEN, DEPTH)
